# Initial kernel scaffold; baseline (speedup 1.0000x reference)
import jax
import jax.numpy as jnp
from jax.experimental import pallas as pl


def kernel(x, edge_index, y, train_mask, W1, b1, W2, b2, W3, b3):
    raise NotImplementedError("write your pallas kernel here")


# speedup vs baseline: 18.8097x; 18.8097x over previous
import functools

import jax
import jax.numpy as jnp
from jax import lax
from jax.experimental import pallas as pl
from jax.experimental.pallas import tpu as pltpu
from jax.experimental.pallas import tpu_sc as plsc

N = 10000
NP = 10240
D = 128
NLBL = 40
E = 320000
NC = 2
NS = 16
NW = NC * NS
CH = 128
EPT = 10240
NCHUNK = EPT // CH
E_PAD = NW * EPT
ROWS_PT = NP // NS
SINK = N
PAD_SPREAD = 240
BLK = 1024
GRID = NP // BLK

_mesh = plsc.VectorSubcoreMesh(core_axis_name="c", subcore_axis_name="s")


@functools.partial(
    pl.kernel,
    out_type=jax.ShapeDtypeStruct((NC, NP), jnp.float32),
    mesh=_mesh,
    scratch_types=[
        pltpu.VMEM((NCHUNK, CH), jnp.int32),
        pltpu.VMEM((CH,), jnp.float32),
        pltpu.VMEM((ROWS_PT,), jnp.float32),
        pltpu.VMEM_SHARED((NP,), jnp.float32),
    ],
)
def _deg_call(dst_hbm, deg_out, dst_v, ones_v, zeros_v, acc):
    c = lax.axis_index("c")
    s = lax.axis_index("s")
    wid = c * NS + s

    def _fill(i, _):
        ones_v[pl.ds(i * 16, 16)] = jnp.ones((16,), jnp.float32)
        return 0

    lax.fori_loop(0, CH // 16, _fill, 0)

    def _fillz(i, _):
        zeros_v[pl.ds(i * 16, 16)] = jnp.zeros((16,), jnp.float32)
        return 0

    lax.fori_loop(0, ROWS_PT // 16, _fillz, 0)

    pltpu.sync_copy(zeros_v, acc.at[pl.ds(s * ROWS_PT, ROWS_PT)])
    pltpu.sync_copy(dst_hbm.at[wid], dst_v)
    plsc.subcore_barrier()

    def _chunk(j, _):
        pltpu.sync_copy(ones_v, acc.at[dst_v.at[j]], add=True)
        return 0

    lax.fori_loop(0, NCHUNK, _chunk, 0)
    plsc.subcore_barrier()
    pltpu.sync_copy(acc.at[pl.ds(s * ROWS_PT, ROWS_PT)],
                    deg_out.at[c, pl.ds(s * ROWS_PT, ROWS_PT)])


@functools.partial(
    pl.kernel,
    out_type=jax.ShapeDtypeStruct((NC, NP, D), jnp.float32),
    mesh=_mesh,
    scratch_types=[
        pltpu.VMEM((NCHUNK, CH), jnp.int32),
        pltpu.VMEM((NCHUNK, CH), jnp.int32),
        pltpu.VMEM((CH, D), jnp.float32),
        pltpu.VMEM_SHARED((NP, D), jnp.float32),
        pltpu.SemaphoreType.DMA,
    ],
)
def _prop_call(zs_hbm, src_hbm, dst_hbm, zeros_hbm, out_hbm,
               src_v, dst_v, rows_v, acc, sem):
    c = lax.axis_index("c")
    s = lax.axis_index("s")
    wid = c * NS + s

    pltpu.sync_copy(zeros_hbm, acc.at[pl.ds(s * ROWS_PT, ROWS_PT)])
    pltpu.sync_copy(src_hbm.at[wid], src_v)
    pltpu.sync_copy(dst_hbm.at[wid], dst_v)
    plsc.subcore_barrier()

    def _chunk(j, _):
        pltpu.async_copy(zs_hbm.at[src_v.at[j]], rows_v, sem).wait()
        pltpu.sync_copy(rows_v, acc.at[dst_v.at[j]], add=True)
        return 0

    lax.fori_loop(0, NCHUNK, _chunk, 0)
    plsc.subcore_barrier()
    pltpu.sync_copy(acc.at[pl.ds(s * ROWS_PT, ROWS_PT)],
                    out_hbm.at[c, pl.ds(s * ROWS_PT, ROWS_PT)])


def _tc1_body(hist_ref, x_ref, w1_ref, dinv_ref, zs1_ref):
    deg = jnp.sum(hist_ref[...], axis=1, keepdims=True) + 1.0
    dinv = lax.rsqrt(deg)
    dinv_ref[...] = dinv
    zs1_ref[...] = dinv * jnp.dot(x_ref[...], w1_ref[...],
                                  preferred_element_type=jnp.float32)


_tc1 = pl.pallas_call(
    _tc1_body,
    grid=(GRID,),
    in_specs=[
        pl.BlockSpec((BLK, NC), lambda g: (g, 0)),
        pl.BlockSpec((BLK, D), lambda g: (g, 0)),
        pl.BlockSpec((D, D), lambda g: (0, 0)),
    ],
    out_specs=[
        pl.BlockSpec((BLK, 1), lambda g: (g, 0)),
        pl.BlockSpec((BLK, D), lambda g: (g, 0)),
    ],
    out_shape=[
        jax.ShapeDtypeStruct((NP, 1), jnp.float32),
        jax.ShapeDtypeStruct((NP, D), jnp.float32),
    ],
)


def _tc2_body(part_ref, zs_ref, dinv_ref, w_ref, b_ref, out_ref):
    dinv = dinv_ref[...]
    h = dinv * (part_ref[0] + part_ref[1] + zs_ref[...]) + b_ref[...]
    h = jnp.maximum(h, 0.0)
    out_ref[...] = dinv * jnp.dot(h, w_ref[...],
                                  preferred_element_type=jnp.float32)


_tc2 = pl.pallas_call(
    _tc2_body,
    grid=(GRID,),
    in_specs=[
        pl.BlockSpec((NC, BLK, D), lambda g: (0, g, 0)),
        pl.BlockSpec((BLK, D), lambda g: (g, 0)),
        pl.BlockSpec((BLK, 1), lambda g: (g, 0)),
        pl.BlockSpec((D, D), lambda g: (0, 0)),
        pl.BlockSpec((1, D), lambda g: (0, 0)),
    ],
    out_specs=pl.BlockSpec((BLK, D), lambda g: (g, 0)),
    out_shape=jax.ShapeDtypeStruct((NP, D), jnp.float32),
)


def _tc3_body(part_ref, zs_ref, dinv_ref, b_ref, out_ref):
    dinv = dinv_ref[...]
    h = dinv * (part_ref[0] + part_ref[1] + zs_ref[...]) + b_ref[...]
    out_ref[...] = dinv * jnp.maximum(h, 0.0)


_tc3 = pl.pallas_call(
    _tc3_body,
    grid=(GRID,),
    in_specs=[
        pl.BlockSpec((NC, BLK, D), lambda g: (0, g, 0)),
        pl.BlockSpec((BLK, D), lambda g: (g, 0)),
        pl.BlockSpec((BLK, 1), lambda g: (g, 0)),
        pl.BlockSpec((1, D), lambda g: (0, 0)),
    ],
    out_specs=pl.BlockSpec((BLK, D), lambda g: (g, 0)),
    out_shape=jax.ShapeDtypeStruct((NP, D), jnp.float32),
)


def _tc4_body(part_ref, zs_ref, dinv_ref, w3_ref, b3_ref, y_ref, m_ref,
              loss_ref, num_acc, den_acc):
    g = dinv_ref[...] * (part_ref[0] + part_ref[1] + zs_ref[...])
    out = jnp.dot(g, w3_ref[...], preferred_element_type=jnp.float32) + b3_ref[...]
    mx = jnp.max(out, axis=1, keepdims=True)
    lse = jnp.log(jnp.sum(jnp.exp(out - mx), axis=1, keepdims=True)) + mx
    y = y_ref[...]
    onehot = lax.broadcasted_iota(jnp.int32, out.shape, 1) == y
    picked = jnp.sum(jnp.where(onehot, out, 0.0), axis=1, keepdims=True)
    valid = m_ref[...] * jnp.where(y != -1, 1.0, 0.0)
    per_node = (lse - picked) * valid

    i = pl.program_id(0)

    @pl.when(i == 0)
    def _():
        num_acc[0] = 0.0
        den_acc[0] = 0.0

    num_acc[0] += jnp.sum(per_node)
    den_acc[0] += jnp.sum(valid)

    @pl.when(i == pl.num_programs(0) - 1)
    def _():
        loss_ref[...] = jnp.reshape(num_acc[0] / jnp.maximum(den_acc[0], 1.0),
                                    (1, 1))


_tc4 = pl.pallas_call(
    _tc4_body,
    grid=(GRID,),
    in_specs=[
        pl.BlockSpec((NC, BLK, D), lambda g: (0, g, 0)),
        pl.BlockSpec((BLK, D), lambda g: (g, 0)),
        pl.BlockSpec((BLK, 1), lambda g: (g, 0)),
        pl.BlockSpec((D, NLBL), lambda g: (0, 0)),
        pl.BlockSpec((1, NLBL), lambda g: (0, 0)),
        pl.BlockSpec((BLK, 1), lambda g: (g, 0)),
        pl.BlockSpec((BLK, 1), lambda g: (g, 0)),
    ],
    out_specs=pl.BlockSpec((1, 1), lambda g: (0, 0)),
    out_shape=jax.ShapeDtypeStruct((1, 1), jnp.float32),
    scratch_shapes=[
        pltpu.SMEM((1,), jnp.float32),
        pltpu.SMEM((1,), jnp.float32),
    ],
)


def kernel(x, edge_index, y, train_mask, W1, b1, W2, b2, W3, b3):
    src = edge_index[0].astype(jnp.int32)
    dst = edge_index[1].astype(jnp.int32)
    pad = SINK + (jnp.arange(E_PAD - E, dtype=jnp.int32) % PAD_SPREAD)
    src_p = jnp.concatenate([src, pad]).reshape(NW, NCHUNK, CH)
    dst_p = jnp.concatenate([dst, pad]).reshape(NW, NCHUNK, CH)

    deg_part = _deg_call(dst_p)
    hist_t = deg_part.T
    x_pad = jnp.pad(x, ((0, NP - N), (0, 0)))
    dinv, zs1 = _tc1(hist_t, x_pad, W1)

    zeros_blk = jnp.zeros((ROWS_PT, D), jnp.float32)
    s1 = _prop_call(zs1, src_p, dst_p, zeros_blk)
    zs2 = _tc2(s1, zs1, dinv, W2, b1.reshape(1, D))
    s2 = _prop_call(zs2, src_p, dst_p, zeros_blk)
    zs3 = _tc3(s2, zs2, dinv, b2.reshape(1, D))
    s3 = _prop_call(zs3, src_p, dst_p, zeros_blk)

    y_pad = jnp.pad(y.astype(jnp.int32), (0, NP - N)).reshape(NP, 1)
    m_pad = jnp.pad(train_mask.astype(jnp.float32), (0, NP - N)).reshape(NP, 1)
    loss = _tc4(s3, zs3, dinv, W3, b3.reshape(1, NLBL), y_pad, m_pad)
    return loss[0, 0]


# speedup vs baseline: 26.4666x; 1.4071x over previous
import functools

import jax
import jax.numpy as jnp
from jax import lax
from jax.experimental import pallas as pl
from jax.experimental.pallas import tpu as pltpu
from jax.experimental.pallas import tpu_sc as plsc

N = 10000
NP = 10240
D = 128
NLBL = 40
E = 320000
NC = 2
NS = 16
NW = NC * NS
CH = 128
EPT = 10240
NCHUNK = EPT // CH
GC = 16
NGROUP = NCHUNK // GC
E_PAD = NW * EPT
ROWS_PT = NP // NS
SINK = N
PAD_SPREAD = 240
BLK = 1024
GRID = NP // BLK

_mesh = plsc.VectorSubcoreMesh(core_axis_name="c", subcore_axis_name="s")


@functools.partial(
    pl.kernel,
    out_type=jax.ShapeDtypeStruct((NC, NP), jnp.float32),
    mesh=_mesh,
    scratch_types=[
        pltpu.VMEM((NCHUNK, CH), jnp.int32),
        pltpu.VMEM((CH,), jnp.float32),
        pltpu.VMEM((ROWS_PT,), jnp.float32),
        pltpu.VMEM_SHARED((NP,), jnp.float32),
    ],
)
def _deg_call(dst_hbm, deg_out, dst_v, ones_v, zeros_v, acc):
    c = lax.axis_index("c")
    s = lax.axis_index("s")
    wid = c * NS + s

    def _fill(i, _):
        ones_v[pl.ds(i * 16, 16)] = jnp.ones((16,), jnp.float32)
        return 0

    lax.fori_loop(0, CH // 16, _fill, 0)

    def _fillz(i, _):
        zeros_v[pl.ds(i * 16, 16)] = jnp.zeros((16,), jnp.float32)
        return 0

    lax.fori_loop(0, ROWS_PT // 16, _fillz, 0)

    pltpu.sync_copy(zeros_v, acc.at[pl.ds(s * ROWS_PT, ROWS_PT)])
    pltpu.sync_copy(dst_hbm.at[wid], dst_v)
    plsc.subcore_barrier()

    def _chunk(j, _):
        pltpu.sync_copy(ones_v, acc.at[dst_v.at[j]], add=True)
        return 0

    lax.fori_loop(0, NCHUNK, _chunk, 0)
    plsc.subcore_barrier()
    pltpu.sync_copy(acc.at[pl.ds(s * ROWS_PT, ROWS_PT)],
                    deg_out.at[c, pl.ds(s * ROWS_PT, ROWS_PT)])


@functools.partial(
    pl.kernel,
    out_type=jax.ShapeDtypeStruct((NC, NP, D), jnp.float32),
    mesh=_mesh,
    scratch_types=[
        pltpu.VMEM((2, GC, CH), jnp.int32),
        pltpu.VMEM((2, GC, CH), jnp.int32),
        pltpu.VMEM((CH, D), jnp.float32),
        pltpu.VMEM((CH, D), jnp.float32),
        pltpu.VMEM_SHARED((NP, D), jnp.float32),
        pltpu.SemaphoreType.DMA,
        pltpu.SemaphoreType.DMA,
        pltpu.SemaphoreType.DMA,
    ],
)
def _prop_call(zs_hbm, src_hbm, dst_hbm, zeros_hbm, out_hbm,
               src_v, dst_v, rows_a, rows_b, acc, sem_a, sem_b, sem_i):
    c = lax.axis_index("c")
    s = lax.axis_index("s")
    wid = c * NS + s

    pltpu.sync_copy(zeros_hbm, acc.at[pl.ds(s * ROWS_PT, ROWS_PT)])
    pltpu.sync_copy(src_hbm.at[wid, pl.ds(0, GC)], src_v.at[0])
    pltpu.sync_copy(dst_hbm.at[wid, pl.ds(0, GC)], dst_v.at[0])
    plsc.subcore_barrier()

    pltpu.async_copy(zs_hbm.at[src_v.at[0, 0]], rows_a, sem_a)
    for g in range(NGROUP):
        slot = g % 2
        nxt = 1 - slot
        if g + 1 < NGROUP:
            pltpu.async_copy(src_hbm.at[wid, pl.ds((g + 1) * GC, GC)],
                             src_v.at[nxt], sem_i)
            pltpu.async_copy(dst_hbm.at[wid, pl.ds((g + 1) * GC, GC)],
                             dst_v.at[nxt], sem_i)

        def _pair(p, _, slot=slot):
            j = 2 * p
            pltpu.async_copy(zs_hbm.at[src_v.at[slot, j + 1]], rows_b, sem_b)
            pltpu.make_async_copy(zs_hbm.at[src_v.at[slot, j]],
                                  rows_a, sem_a).wait()
            pltpu.sync_copy(rows_a, acc.at[dst_v.at[slot, j]], add=True)

            @pl.when(p + 1 < GC // 2)
            def _():
                pltpu.async_copy(zs_hbm.at[src_v.at[slot, j + 2]],
                                 rows_a, sem_a)

            pltpu.make_async_copy(zs_hbm.at[src_v.at[slot, j + 1]],
                                  rows_b, sem_b).wait()
            pltpu.sync_copy(rows_b, acc.at[dst_v.at[slot, j + 1]], add=True)
            return 0

        lax.fori_loop(0, GC // 2, _pair, 0)
        if g + 1 < NGROUP:
            pltpu.make_async_copy(src_hbm.at[wid, pl.ds((g + 1) * GC, GC)],
                                  src_v.at[nxt], sem_i).wait()
            pltpu.make_async_copy(dst_hbm.at[wid, pl.ds((g + 1) * GC, GC)],
                                  dst_v.at[nxt], sem_i).wait()
            pltpu.async_copy(zs_hbm.at[src_v.at[nxt, 0]], rows_a, sem_a)
    plsc.subcore_barrier()
    pltpu.sync_copy(acc.at[pl.ds(s * ROWS_PT, ROWS_PT)],
                    out_hbm.at[c, pl.ds(s * ROWS_PT, ROWS_PT)])


def _tc1_body(hist_ref, x_ref, w1_ref, dinv_ref, zs1_ref):
    deg = jnp.sum(hist_ref[...], axis=1, keepdims=True) + 1.0
    dinv = lax.rsqrt(deg)
    dinv_ref[...] = dinv
    zs1_ref[...] = dinv * jnp.dot(x_ref[...], w1_ref[...],
                                  preferred_element_type=jnp.float32)


_tc1 = pl.pallas_call(
    _tc1_body,
    grid=(GRID,),
    in_specs=[
        pl.BlockSpec((BLK, NC), lambda g: (g, 0)),
        pl.BlockSpec((BLK, D), lambda g: (g, 0)),
        pl.BlockSpec((D, D), lambda g: (0, 0)),
    ],
    out_specs=[
        pl.BlockSpec((BLK, 1), lambda g: (g, 0)),
        pl.BlockSpec((BLK, D), lambda g: (g, 0)),
    ],
    out_shape=[
        jax.ShapeDtypeStruct((NP, 1), jnp.float32),
        jax.ShapeDtypeStruct((NP, D), jnp.float32),
    ],
)


def _tc2_body(part_ref, zs_ref, dinv_ref, w_ref, b_ref, out_ref):
    dinv = dinv_ref[...]
    h = dinv * (part_ref[0] + part_ref[1] + zs_ref[...]) + b_ref[...]
    h = jnp.maximum(h, 0.0)
    out_ref[...] = dinv * jnp.dot(h, w_ref[...],
                                  preferred_element_type=jnp.float32)


_tc2 = pl.pallas_call(
    _tc2_body,
    grid=(GRID,),
    in_specs=[
        pl.BlockSpec((NC, BLK, D), lambda g: (0, g, 0)),
        pl.BlockSpec((BLK, D), lambda g: (g, 0)),
        pl.BlockSpec((BLK, 1), lambda g: (g, 0)),
        pl.BlockSpec((D, D), lambda g: (0, 0)),
        pl.BlockSpec((1, D), lambda g: (0, 0)),
    ],
    out_specs=pl.BlockSpec((BLK, D), lambda g: (g, 0)),
    out_shape=jax.ShapeDtypeStruct((NP, D), jnp.float32),
)


def _tc3_body(part_ref, zs_ref, dinv_ref, b_ref, out_ref):
    dinv = dinv_ref[...]
    h = dinv * (part_ref[0] + part_ref[1] + zs_ref[...]) + b_ref[...]
    out_ref[...] = dinv * jnp.maximum(h, 0.0)


_tc3 = pl.pallas_call(
    _tc3_body,
    grid=(GRID,),
    in_specs=[
        pl.BlockSpec((NC, BLK, D), lambda g: (0, g, 0)),
        pl.BlockSpec((BLK, D), lambda g: (g, 0)),
        pl.BlockSpec((BLK, 1), lambda g: (g, 0)),
        pl.BlockSpec((1, D), lambda g: (0, 0)),
    ],
    out_specs=pl.BlockSpec((BLK, D), lambda g: (g, 0)),
    out_shape=jax.ShapeDtypeStruct((NP, D), jnp.float32),
)


def _tc4_body(part_ref, zs_ref, dinv_ref, w3_ref, b3_ref, y_ref, m_ref,
              loss_ref, num_acc, den_acc):
    g = dinv_ref[...] * (part_ref[0] + part_ref[1] + zs_ref[...])
    out = jnp.dot(g, w3_ref[...], preferred_element_type=jnp.float32) + b3_ref[...]
    mx = jnp.max(out, axis=1, keepdims=True)
    lse = jnp.log(jnp.sum(jnp.exp(out - mx), axis=1, keepdims=True)) + mx
    y = y_ref[...]
    onehot = lax.broadcasted_iota(jnp.int32, out.shape, 1) == y
    picked = jnp.sum(jnp.where(onehot, out, 0.0), axis=1, keepdims=True)
    valid = m_ref[...] * jnp.where(y != -1, 1.0, 0.0)
    per_node = (lse - picked) * valid

    i = pl.program_id(0)

    @pl.when(i == 0)
    def _():
        num_acc[0] = 0.0
        den_acc[0] = 0.0

    num_acc[0] += jnp.sum(per_node)
    den_acc[0] += jnp.sum(valid)

    @pl.when(i == pl.num_programs(0) - 1)
    def _():
        loss_ref[...] = jnp.reshape(num_acc[0] / jnp.maximum(den_acc[0], 1.0),
                                    (1, 1))


_tc4 = pl.pallas_call(
    _tc4_body,
    grid=(GRID,),
    in_specs=[
        pl.BlockSpec((NC, BLK, D), lambda g: (0, g, 0)),
        pl.BlockSpec((BLK, D), lambda g: (g, 0)),
        pl.BlockSpec((BLK, 1), lambda g: (g, 0)),
        pl.BlockSpec((D, NLBL), lambda g: (0, 0)),
        pl.BlockSpec((1, NLBL), lambda g: (0, 0)),
        pl.BlockSpec((BLK, 1), lambda g: (g, 0)),
        pl.BlockSpec((BLK, 1), lambda g: (g, 0)),
    ],
    out_specs=pl.BlockSpec((1, 1), lambda g: (0, 0)),
    out_shape=jax.ShapeDtypeStruct((1, 1), jnp.float32),
    scratch_shapes=[
        pltpu.SMEM((1,), jnp.float32),
        pltpu.SMEM((1,), jnp.float32),
    ],
)


def kernel(x, edge_index, y, train_mask, W1, b1, W2, b2, W3, b3):
    src = edge_index[0].astype(jnp.int32)
    dst = edge_index[1].astype(jnp.int32)
    pad = SINK + (jnp.arange(E_PAD - E, dtype=jnp.int32) % PAD_SPREAD)
    src_p = jnp.concatenate([src, pad]).reshape(NW, NCHUNK, CH)
    dst_p = jnp.concatenate([dst, pad]).reshape(NW, NCHUNK, CH)

    deg_part = _deg_call(dst_p)
    hist_t = deg_part.T
    x_pad = jnp.pad(x, ((0, NP - N), (0, 0)))
    dinv, zs1 = _tc1(hist_t, x_pad, W1)

    zeros_blk = jnp.zeros((ROWS_PT, D), jnp.float32)
    s1 = _prop_call(zs1, src_p, dst_p, zeros_blk)
    zs2 = _tc2(s1, zs1, dinv, W2, b1.reshape(1, D))
    s2 = _prop_call(zs2, src_p, dst_p, zeros_blk)
    zs3 = _tc3(s2, zs2, dinv, b2.reshape(1, D))
    s3 = _prop_call(zs3, src_p, dst_p, zeros_blk)

    y_pad = jnp.pad(y.astype(jnp.int32), (0, NP - N)).reshape(NP, 1)
    m_pad = jnp.pad(train_mask.astype(jnp.float32), (0, NP - N)).reshape(NP, 1)
    loss = _tc4(s3, zs3, dinv, W3, b3.reshape(1, NLBL), y_pad, m_pad)
    return loss[0, 0]
